# baseline (speedup 1.0000x reference)
import functools

import jax
import jax.numpy as jnp
from jax import lax
from jax.experimental import pallas as pl
from jax.experimental.pallas import tpu as pltpu
from jax.experimental.pallas import tpu_sc as plsc

DIM = 128
N_NEGS = 32
ALPHA = 1.0
DECAY = 1e-4
LANES = 16
NCHUNKS = DIM // LANES

NC = 2
NS = 16
NW = NC * NS

CHUNK = 64
GROUP = 4
NBUF = 4
NGR = CHUNK // GROUP


def _lane_perm(x, idx):
    return lax.gather(
        x, idx[:, None],
        lax.GatherDimensionNumbers(
            offset_dims=(), collapsed_slice_dims=(0,), start_index_map=(0,)),
        (1,), mode=lax.GatherScatterMode.PROMISE_IN_BOUNDS)


def _sc_body(B, user_hbm, item_hbm, users_hbm, pos_hbm, neg_hbm,
             ox_hbm, oss_hbm,
             u_idx, p_idx, n_idx, s_buf, p_buf, neg_buf,
             o_x, o_ss,
             sem_s, sem_p, sem_i, sem_o, sem0, sem1, sem2, sem3):
    b_per_w = B // NW
    n_chunks = b_per_w // CHUNK
    total_groups = n_chunks * NGR
    wid = lax.axis_index("s") * NC + lax.axis_index("c")
    wbase = wid * b_per_w
    ring = (sem0, sem1, sem2, sem3)

    def stage_idx(chunk, q):
        base = wbase + chunk * CHUNK
        pltpu.async_copy(users_hbm.at[pl.ds(base, CHUNK)], u_idx.at[q], sem_i)
        pltpu.async_copy(pos_hbm.at[pl.ds(base, CHUNK)], p_idx.at[q], sem_i)
        pltpu.async_copy(neg_hbm.at[pl.ds(base * N_NEGS, CHUNK * N_NEGS)],
                         n_idx.at[q], sem_i)

    def wait_idx():
        pltpu.make_async_copy(
            users_hbm.at[pl.ds(0, CHUNK)], u_idx.at[0], sem_i).wait()
        pltpu.make_async_copy(
            pos_hbm.at[pl.ds(0, CHUNK)], p_idx.at[0], sem_i).wait()
        pltpu.make_async_copy(
            neg_hbm.at[pl.ds(0, CHUNK * N_NEGS)], n_idx.at[0], sem_i).wait()

    def stage_rows(q):
        pltpu.async_copy(user_hbm.at[u_idx.at[q]], s_buf.at[q], sem_s)
        pltpu.async_copy(item_hbm.at[p_idx.at[q]], p_buf.at[q], sem_p)

    def wait_rows():
        pltpu.make_async_copy(
            user_hbm.at[pl.ds(0, CHUNK)], s_buf.at[0], sem_s).wait()
        pltpu.make_async_copy(
            item_hbm.at[pl.ds(0, CHUNK)], p_buf.at[0], sem_p).wait()

    def issue_neg(g_global, slot):
        q = (g_global // NGR) % 2
        off = (g_global % NGR) * (GROUP * N_NEGS)
        pltpu.async_copy(
            item_hbm.at[n_idx.at[q, pl.ds(off, GROUP * N_NEGS)]],
            neg_buf.at[slot], ring[slot])

    stage_idx(0, 0)
    wait_idx()
    stage_rows(0)
    for slot in range(NBUF):
        issue_neg(slot, slot)
    wait_rows()

    def per_block(blk, carry):
        for b in range(NBUF):
            G = blk * NBUF + b
            chunk = G // NGR
            local = G % NGR
            q = chunk % 2

            @pl.when(jnp.logical_and(local == 0, chunk > 0))
            def _():
                wait_rows()

            @pl.when(jnp.logical_and(local == 0, chunk >= 2))
            def _():
                pltpu.make_async_copy(
                    o_x.at[0], ox_hbm.at[pl.ds(0, CHUNK)], sem_o).wait()

            @pl.when(jnp.logical_and(local == 0, chunk + 1 < n_chunks))
            def _():
                stage_idx(chunk + 1, (chunk + 1) % 2)

            @pl.when(jnp.logical_and(local == NGR // 2, chunk + 1 < n_chunks))
            def _():
                wait_idx()
                stage_rows((chunk + 1) % 2)

            pltpu.make_async_copy(
                item_hbm.at[pl.ds(0, GROUP * N_NEGS)], neg_buf.at[b],
                ring[b]).wait()

            def per_b(i, carry, buf=b, local=local, q=q):
                row = local * GROUP + i
                iota = lax.iota(jnp.int32, LANES)
                s = [s_buf[q, row, pl.ds(c * LANES, LANES)]
                     for c in range(NCHUNKS)]
                best_s = jnp.full((LANES,), -jnp.inf, jnp.float32)
                best_j = jnp.zeros((LANES,), jnp.int32)
                for j in range(N_NEGS):
                    r = i * N_NEGS + j
                    prod = [s[c] * neg_buf[buf, r, pl.ds(c * LANES, LANES)]
                            for c in range(NCHUNKS)]
                    while len(prod) > 1:
                        prod = [prod[k] + prod[k + 1]
                                for k in range(0, len(prod) - 1, 2)] \
                            + ([prod[-1]] if len(prod) % 2 else [])
                    acc = prod[0]
                    for k in (8, 4, 2, 1):
                        acc = acc + _lane_perm(acc, iota ^ k)
                    better = acc > best_s
                    best_j = jnp.where(
                        better, jnp.full((LANES,), j, jnp.int32), best_j)
                    best_s = jnp.maximum(best_s, acc)
                rbest = i * N_NEGS + best_j[0]
                ssu_acc, ssp_acc, ssn_acc = carry
                x_acc = jnp.zeros((LANES,), jnp.float32)
                for c in range(NCHUNKS):
                    h = neg_buf[buf, rbest, pl.ds(c * LANES, LANES)]
                    p = p_buf[q, row, pl.ds(c * LANES, LANES)]
                    hp = h - p
                    m = s[c] * hp
                    nw = 1.0 / (1.0 + ALPHA * jnp.exp(0.0 - m))
                    ne = p + nw * hp
                    x_acc = x_acc - m * nw
                    ssu_acc = ssu_acc + s[c] * s[c]
                    ssp_acc = ssp_acc + p * p
                    ssn_acc = ssn_acc + ne * ne
                o_x[q, row] = x_acc
                return (ssu_acc, ssp_acc, ssn_acc)

            carry = lax.fori_loop(0, GROUP, per_b, carry)

            @pl.when(local == NGR - 1)
            def _():
                base = wbase + chunk * CHUNK
                pltpu.async_copy(
                    o_x.at[q], ox_hbm.at[pl.ds(base, CHUNK)], sem_o)

            @pl.when(G + NBUF < total_groups)
            def _():
                issue_neg(G + NBUF, b)
        return carry

    zero = jnp.zeros((LANES,), jnp.float32)
    ssu, ssp, ssn = lax.fori_loop(
        0, total_groups // NBUF, per_block, (zero, zero, zero))
    for _ in range(2):
        pltpu.make_async_copy(
            o_x.at[0], ox_hbm.at[pl.ds(0, CHUNK)], sem_o).wait()
    o_ss[0] = ssu
    o_ss[1] = ssp
    o_ss[2] = ssn
    pltpu.sync_copy(o_ss, oss_hbm.at[wid])


def _finalize_body(B, x_ref, ss_ref, out_ref):
    x = jnp.sum(x_ref[...], axis=1)
    sig = 1.0 / (1.0 + jnp.exp(-x))
    bpr = jnp.mean(-jnp.log(1e-5 + sig))
    ss = jnp.sum(ss_ref[...], axis=(0, 2))
    reg = DECAY * (jnp.sqrt(ss[0]) + jnp.sqrt(ss[1]) + jnp.sqrt(ss[2])) / B
    out_ref[...] = jnp.reshape(bpr + reg / B, (1, 1))


def kernel(user_emb, item_emb, users, pos_items, neg_items):
    B = users.shape[0]
    users32 = users.astype(jnp.int32)
    pos32 = pos_items.astype(jnp.int32)
    neg32 = neg_items.astype(jnp.int32).reshape(-1)

    mesh = plsc.VectorSubcoreMesh(core_axis_name="c", subcore_axis_name="s")
    out_t = [jax.ShapeDtypeStruct((B, LANES), jnp.float32),
             jax.ShapeDtypeStruct((NW, 3, LANES), jnp.float32)]
    sc = pl.kernel(
        functools.partial(_sc_body, B),
        out_type=out_t,
        mesh=mesh,
        scratch_types=[
            pltpu.VMEM((2, CHUNK), jnp.int32),
            pltpu.VMEM((2, CHUNK), jnp.int32),
            pltpu.VMEM((2, CHUNK * N_NEGS), jnp.int32),
            pltpu.VMEM((2, CHUNK, DIM), jnp.float32),
            pltpu.VMEM((2, CHUNK, DIM), jnp.float32),
            pltpu.VMEM((NBUF, GROUP * N_NEGS, DIM), jnp.float32),
            pltpu.VMEM((2, CHUNK, LANES), jnp.float32),
            pltpu.VMEM((3, LANES), jnp.float32),
            pltpu.SemaphoreType.DMA,
            pltpu.SemaphoreType.DMA,
            pltpu.SemaphoreType.DMA,
            pltpu.SemaphoreType.DMA,
            pltpu.SemaphoreType.DMA,
            pltpu.SemaphoreType.DMA,
            pltpu.SemaphoreType.DMA,
            pltpu.SemaphoreType.DMA,
        ],
    )
    x_s, ss_s = sc(user_emb, item_emb, users32, pos32, neg32)

    loss2d = pl.pallas_call(
        functools.partial(_finalize_body, B),
        out_shape=jax.ShapeDtypeStruct((1, 1), jnp.float32),
    )(x_s, ss_s)
    loss = loss2d.reshape(())
    return (loss, loss, loss)


# speedup vs baseline: 1.0229x; 1.0229x over previous
import functools

import jax
import jax.numpy as jnp
from jax import lax
from jax.experimental import pallas as pl
from jax.experimental.pallas import tpu as pltpu
from jax.experimental.pallas import tpu_sc as plsc

DIM = 128
N_NEGS = 32
ALPHA = 1.0
DECAY = 1e-4
LANES = 16
NCHUNKS = DIM // LANES

NC = 2
NS = 16
NW = NC * NS

CHUNK = 32
GROUP = 4
NBUF = 4
NGR = CHUNK // GROUP


def _lane_perm(x, idx):
    return lax.gather(
        x, idx[:, None],
        lax.GatherDimensionNumbers(
            offset_dims=(), collapsed_slice_dims=(0,), start_index_map=(0,)),
        (1,), mode=lax.GatherScatterMode.PROMISE_IN_BOUNDS)


def _sc_body(B, user_hbm, item_hbm, users_hbm, pos_hbm, neg_hbm,
             ox_hbm, oss_hbm,
             u_idx, p_idx, n_idx, s_buf, p_buf, neg_buf,
             o_x, o_ss,
             sem_s, sem_p, sem_i, sem_o, sem0, sem1, sem2, sem3):
    b_per_w = B // NW
    n_chunks = b_per_w // CHUNK
    total_groups = n_chunks * NGR
    wid = lax.axis_index("s") * NC + lax.axis_index("c")
    wbase = wid * b_per_w
    ring = (sem0, sem1, sem2, sem3)

    def stage_idx(chunk, q):
        base = wbase + chunk * CHUNK
        pltpu.async_copy(users_hbm.at[pl.ds(base, CHUNK)], u_idx.at[q], sem_i)
        pltpu.async_copy(pos_hbm.at[pl.ds(base, CHUNK)], p_idx.at[q], sem_i)
        pltpu.async_copy(neg_hbm.at[pl.ds(base * N_NEGS, CHUNK * N_NEGS)],
                         n_idx.at[q], sem_i)

    def wait_idx():
        pltpu.make_async_copy(
            users_hbm.at[pl.ds(0, CHUNK)], u_idx.at[0], sem_i).wait()
        pltpu.make_async_copy(
            pos_hbm.at[pl.ds(0, CHUNK)], p_idx.at[0], sem_i).wait()
        pltpu.make_async_copy(
            neg_hbm.at[pl.ds(0, CHUNK * N_NEGS)], n_idx.at[0], sem_i).wait()

    def stage_rows(q):
        pltpu.async_copy(user_hbm.at[u_idx.at[q]], s_buf.at[q], sem_s)
        pltpu.async_copy(item_hbm.at[p_idx.at[q]], p_buf.at[q], sem_p)

    def wait_rows():
        pltpu.make_async_copy(
            user_hbm.at[pl.ds(0, CHUNK)], s_buf.at[0], sem_s).wait()
        pltpu.make_async_copy(
            item_hbm.at[pl.ds(0, CHUNK)], p_buf.at[0], sem_p).wait()

    def issue_neg(g_global, slot):
        q = (g_global // NGR) % 2
        off = (g_global % NGR) * (GROUP * N_NEGS)
        pltpu.async_copy(
            item_hbm.at[n_idx.at[q, pl.ds(off, GROUP * N_NEGS)]],
            neg_buf.at[slot], ring[slot])

    stage_idx(0, 0)
    wait_idx()
    stage_rows(0)
    for slot in range(NBUF):
        issue_neg(slot, slot)
    wait_rows()

    def per_block(blk, carry):
        for b in range(NBUF):
            G = blk * NBUF + b
            chunk = G // NGR
            local = G % NGR
            q = chunk % 2

            @pl.when(jnp.logical_and(local == 0, chunk > 0))
            def _():
                wait_rows()

            @pl.when(jnp.logical_and(local == 0, chunk >= 2))
            def _():
                pltpu.make_async_copy(
                    o_x.at[0], ox_hbm.at[pl.ds(0, CHUNK)], sem_o).wait()

            @pl.when(jnp.logical_and(local == 0, chunk + 1 < n_chunks))
            def _():
                stage_idx(chunk + 1, (chunk + 1) % 2)

            @pl.when(jnp.logical_and(local == NGR // 2, chunk + 1 < n_chunks))
            def _():
                wait_idx()
                stage_rows((chunk + 1) % 2)

            pltpu.make_async_copy(
                item_hbm.at[pl.ds(0, GROUP * N_NEGS)], neg_buf.at[b],
                ring[b]).wait()

            def per_b(i, carry, buf=b, local=local, q=q):
                row = local * GROUP + i
                iota = lax.iota(jnp.int32, LANES)
                s = [s_buf[q, row, pl.ds(c * LANES, LANES)]
                     for c in range(NCHUNKS)]
                best_s = jnp.full((LANES,), -jnp.inf, jnp.float32)
                best_j = jnp.zeros((LANES,), jnp.int32)
                for j in range(N_NEGS):
                    r = i * N_NEGS + j
                    prod = [s[c] * neg_buf[buf, r, pl.ds(c * LANES, LANES)]
                            for c in range(NCHUNKS)]
                    while len(prod) > 1:
                        prod = [prod[k] + prod[k + 1]
                                for k in range(0, len(prod) - 1, 2)] \
                            + ([prod[-1]] if len(prod) % 2 else [])
                    acc = prod[0]
                    for k in (8, 4, 2, 1):
                        acc = acc + _lane_perm(acc, iota ^ k)
                    better = acc > best_s
                    best_j = jnp.where(
                        better, jnp.full((LANES,), j, jnp.int32), best_j)
                    best_s = jnp.maximum(best_s, acc)
                rbest = i * N_NEGS + best_j[0]
                ssu_acc, ssp_acc, ssn_acc = carry
                x_acc = jnp.zeros((LANES,), jnp.float32)
                for c in range(NCHUNKS):
                    h = neg_buf[buf, rbest, pl.ds(c * LANES, LANES)]
                    p = p_buf[q, row, pl.ds(c * LANES, LANES)]
                    hp = h - p
                    m = s[c] * hp
                    nw = 1.0 / (1.0 + ALPHA * jnp.exp(0.0 - m))
                    ne = p + nw * hp
                    x_acc = x_acc - m * nw
                    ssu_acc = ssu_acc + s[c] * s[c]
                    ssp_acc = ssp_acc + p * p
                    ssn_acc = ssn_acc + ne * ne
                o_x[q, row] = x_acc
                return (ssu_acc, ssp_acc, ssn_acc)

            carry = lax.fori_loop(0, GROUP, per_b, carry)

            @pl.when(local == NGR - 1)
            def _():
                base = wbase + chunk * CHUNK
                pltpu.async_copy(
                    o_x.at[q], ox_hbm.at[pl.ds(base, CHUNK)], sem_o)

            @pl.when(G + NBUF < total_groups)
            def _():
                issue_neg(G + NBUF, b)
        return carry

    zero = jnp.zeros((LANES,), jnp.float32)
    ssu, ssp, ssn = lax.fori_loop(
        0, total_groups // NBUF, per_block, (zero, zero, zero))
    for _ in range(2):
        pltpu.make_async_copy(
            o_x.at[0], ox_hbm.at[pl.ds(0, CHUNK)], sem_o).wait()
    o_ss[0] = ssu
    o_ss[1] = ssp
    o_ss[2] = ssn
    pltpu.sync_copy(o_ss, oss_hbm.at[wid])


def _finalize_body(B, x_ref, ss_ref, out_ref):
    x = jnp.sum(x_ref[...], axis=1)
    sig = 1.0 / (1.0 + jnp.exp(-x))
    bpr = jnp.mean(-jnp.log(1e-5 + sig))
    ss = jnp.sum(ss_ref[...], axis=(0, 2))
    reg = DECAY * (jnp.sqrt(ss[0]) + jnp.sqrt(ss[1]) + jnp.sqrt(ss[2])) / B
    out_ref[...] = jnp.reshape(bpr + reg / B, (1, 1))


def kernel(user_emb, item_emb, users, pos_items, neg_items):
    B = users.shape[0]
    users32 = users.astype(jnp.int32)
    pos32 = pos_items.astype(jnp.int32)
    neg32 = neg_items.astype(jnp.int32).reshape(-1)

    mesh = plsc.VectorSubcoreMesh(core_axis_name="c", subcore_axis_name="s")
    out_t = [jax.ShapeDtypeStruct((B, LANES), jnp.float32),
             jax.ShapeDtypeStruct((NW, 3, LANES), jnp.float32)]
    sc = pl.kernel(
        functools.partial(_sc_body, B),
        out_type=out_t,
        mesh=mesh,
        scratch_types=[
            pltpu.VMEM((2, CHUNK), jnp.int32),
            pltpu.VMEM((2, CHUNK), jnp.int32),
            pltpu.VMEM((2, CHUNK * N_NEGS), jnp.int32),
            pltpu.VMEM((2, CHUNK, DIM), jnp.float32),
            pltpu.VMEM((2, CHUNK, DIM), jnp.float32),
            pltpu.VMEM((NBUF, GROUP * N_NEGS, DIM), jnp.float32),
            pltpu.VMEM((2, CHUNK, LANES), jnp.float32),
            pltpu.VMEM((3, LANES), jnp.float32),
            pltpu.SemaphoreType.DMA,
            pltpu.SemaphoreType.DMA,
            pltpu.SemaphoreType.DMA,
            pltpu.SemaphoreType.DMA,
            pltpu.SemaphoreType.DMA,
            pltpu.SemaphoreType.DMA,
            pltpu.SemaphoreType.DMA,
            pltpu.SemaphoreType.DMA,
        ],
    )
    x_s, ss_s = sc(user_emb, item_emb, users32, pos32, neg32)

    loss2d = pl.pallas_call(
        functools.partial(_finalize_body, B),
        out_shape=jax.ShapeDtypeStruct((1, 1), jnp.float32),
    )(x_s, ss_s)
    loss = loss2d.reshape(())
    return (loss, loss, loss)
